# baseline (speedup 1.0000x reference)
import functools

import jax
import jax.numpy as jnp
from jax import lax
from jax.experimental import pallas as pl
from jax.experimental.pallas import tpu as pltpu
from jax.experimental.pallas import tpu_sc as plsc

N = 100000
H = 128

NC = 2
NS = 16
NW = NC * NS
CHUNK = 128
CHUNKS_PER_W = 25
ROWS_PER_W = CHUNK * CHUNKS_PER_W
NPAD = NW * ROWS_PER_W


def _embed_sc(idx_grid, table):
    mesh = plsc.VectorSubcoreMesh(core_axis_name="c", subcore_axis_name="s")

    @functools.partial(
        pl.kernel,
        out_type=jax.ShapeDtypeStruct((NPAD, H), jnp.float32),
        mesh=mesh,
        scratch_types=[
            pltpu.VMEM((CHUNKS_PER_W, CHUNK), jnp.int32),
            pltpu.VMEM((CHUNK, H), jnp.float32),
            pltpu.SemaphoreType.DMA,
        ],
    )
    def k(idx_hbm, table_hbm, out_hbm, idx_v, rows_v, sem):
        wid = lax.axis_index("s") * NC + lax.axis_index("c")
        pltpu.sync_copy(idx_hbm.at[wid], idx_v)
        base = wid * ROWS_PER_W

        def body(j, carry):
            pltpu.async_copy(table_hbm.at[idx_v.at[j]], rows_v, sem).wait()
            pltpu.sync_copy(rows_v, out_hbm.at[pl.ds(base + j * CHUNK, CHUNK)])
            return carry

        lax.fori_loop(0, CHUNKS_PER_W, body, 0)

    return k(idx_grid, table)


def kernel(atomic_numbers, pos, batch, edge_index, cell, cell_offsets,
           neighbors, W):
    z = atomic_numbers.astype(jnp.int32)
    z_pad = jnp.concatenate([z, jnp.zeros((NPAD - N,), jnp.int32)])
    idx_grid = z_pad.reshape(NW, CHUNKS_PER_W, CHUNK)
    h = _embed_sc(idx_grid, W.astype(jnp.float32))[:N]
    return (h, atomic_numbers, pos, batch, edge_index, cell, cell_offsets,
            neighbors)


# speedup vs baseline: 2.0062x; 2.0062x over previous
import functools

import jax
import jax.numpy as jnp
from jax import lax
from jax.experimental import pallas as pl
from jax.experimental.pallas import tpu as pltpu
from jax.experimental.pallas import tpu_sc as plsc

N = 100000
H = 128

NC = 2
NS = 16
NW = NC * NS
CHUNK = 128
CPW = 25
NBUF = 5
LAST = N - CHUNK


def _embed_sc(z, table):
    mesh = plsc.VectorSubcoreMesh(core_axis_name="c", subcore_axis_name="s")

    @functools.partial(
        pl.kernel,
        out_type=jax.ShapeDtypeStruct((N, H), jnp.float32),
        mesh=mesh,
        scratch_types=[
            pltpu.VMEM((CPW, CHUNK), jnp.int32),
            pltpu.VMEM((NBUF, CHUNK, H), jnp.float32),
            pltpu.SemaphoreType.DMA,
            pltpu.SemaphoreType.DMA((NBUF,)),
            pltpu.SemaphoreType.DMA((NBUF,)),
        ],
    )
    def k(z_hbm, table_hbm, out_hbm, idx_v, rows_v, sem_i, sem_g, sem_w):
        wid = lax.axis_index("s") * NC + lax.axis_index("c")

        def off_of(j):
            return lax.min((wid + j * NW) * CHUNK, LAST)

        def fire_idx(j, c):
            pltpu.async_copy(z_hbm.at[pl.ds(off_of(j), CHUNK)],
                             idx_v.at[j], sem_i)
            return c
        lax.fori_loop(0, CPW, fire_idx, 0)

        def drain_idx(j, c):
            pltpu.make_async_copy(z_hbm.at[pl.ds(0, CHUNK)],
                                  idx_v.at[0], sem_i).wait()
            return c
        lax.fori_loop(0, CPW, drain_idx, 0)

        def gather(j, b):
            pltpu.async_copy(table_hbm.at[idx_v.at[j]], rows_v.at[b],
                             sem_g.at[b])

        def write(j, b):
            pltpu.async_copy(rows_v.at[b],
                             out_hbm.at[pl.ds(off_of(j), CHUNK)],
                             sem_w.at[b])

        def wait_g(b):
            pltpu.make_async_copy(table_hbm.at[idx_v.at[0]], rows_v.at[b],
                                  sem_g.at[b]).wait()

        def wait_w(b):
            pltpu.make_async_copy(rows_v.at[b],
                                  out_hbm.at[pl.ds(0, CHUNK)],
                                  sem_w.at[b]).wait()

        def body(j, c):
            b = lax.rem(j, NBUF)

            @pl.when(j >= NBUF)
            def _():
                wait_w(b)

            gather(j, b)

            @pl.when(j >= 1)
            def _():
                bp = lax.rem(j - 1, NBUF)
                wait_g(bp)
                write(j - 1, bp)
            return c

        lax.fori_loop(0, CPW, body, 0)

        last_b = (CPW - 1) % NBUF
        wait_g(last_b)
        write(CPW - 1, last_b)

        def drain_w(b, c):
            wait_w(b)
            return c
        lax.fori_loop(0, NBUF, drain_w, 0)

    return k(z, table)


def kernel(atomic_numbers, pos, batch, edge_index, cell, cell_offsets,
           neighbors, W):
    z = atomic_numbers.astype(jnp.int32)
    h = _embed_sc(z, W.astype(jnp.float32))
    return (h, atomic_numbers, pos, batch, edge_index, cell, cell_offsets,
            neighbors)


# speedup vs baseline: 2.0399x; 1.0168x over previous
import functools

import jax
import jax.numpy as jnp
from jax import lax
from jax.experimental import pallas as pl
from jax.experimental.pallas import tpu as pltpu
from jax.experimental.pallas import tpu_sc as plsc

N = 100000
H = 128

NC = 2
NS = 16
NW = NC * NS
CHUNK = 128
CPW = 25
NBUF = 7
LAG = 3
LAST = N - CHUNK


def _embed_sc(z, table):
    mesh = plsc.VectorSubcoreMesh(core_axis_name="c", subcore_axis_name="s")

    @functools.partial(
        pl.kernel,
        out_type=jax.ShapeDtypeStruct((N, H), jnp.float32),
        mesh=mesh,
        scratch_types=[
            pltpu.VMEM((CPW, CHUNK), jnp.int32),
            pltpu.VMEM((NBUF, CHUNK, H), jnp.float32),
            pltpu.SemaphoreType.DMA,
            pltpu.SemaphoreType.DMA((NBUF,)),
            pltpu.SemaphoreType.DMA((NBUF,)),
        ],
    )
    def k(z_hbm, table_hbm, out_hbm, idx_v, rows_v, sem_i, sem_g, sem_w):
        wid = lax.axis_index("s") * NC + lax.axis_index("c")

        def off_of(j):
            return lax.min((wid + j * NW) * CHUNK, LAST)

        def fire_idx(j, c):
            pltpu.async_copy(z_hbm.at[pl.ds(off_of(j), CHUNK)],
                             idx_v.at[j], sem_i)
            return c
        lax.fori_loop(0, CPW, fire_idx, 0)

        def drain_idx(j, c):
            pltpu.make_async_copy(z_hbm.at[pl.ds(0, CHUNK)],
                                  idx_v.at[0], sem_i).wait()
            return c
        lax.fori_loop(0, CPW, drain_idx, 0)

        def gather(j, b):
            pltpu.async_copy(table_hbm.at[idx_v.at[j]], rows_v.at[b],
                             sem_g.at[b])

        def write(j, b):
            pltpu.async_copy(rows_v.at[b],
                             out_hbm.at[pl.ds(off_of(j), CHUNK)],
                             sem_w.at[b])

        def wait_g(b):
            pltpu.make_async_copy(table_hbm.at[idx_v.at[0]], rows_v.at[b],
                                  sem_g.at[b]).wait()

        def wait_w(b):
            pltpu.make_async_copy(rows_v.at[b],
                                  out_hbm.at[pl.ds(0, CHUNK)],
                                  sem_w.at[b]).wait()

        def body(j, c):
            b = lax.rem(j, NBUF)

            @pl.when(j >= NBUF)
            def _():
                wait_w(b)

            gather(j, b)

            @pl.when(j >= LAG)
            def _():
                bp = lax.rem(j - LAG, NBUF)
                wait_g(bp)
                write(j - LAG, bp)
            return c

        lax.fori_loop(0, CPW, body, 0)

        def tail(j, c):
            b = lax.rem(j, NBUF)
            wait_g(b)
            write(j, b)
            return c
        lax.fori_loop(CPW - LAG, CPW, tail, 0)

        def drain_w(j, c):
            wait_w(lax.rem(j, NBUF))
            return c
        lax.fori_loop(CPW - NBUF, CPW, drain_w, 0)

    return k(z, table)


def kernel(atomic_numbers, pos, batch, edge_index, cell, cell_offsets,
           neighbors, W):
    z = atomic_numbers.astype(jnp.int32)
    h = _embed_sc(z, W.astype(jnp.float32))
    return (h, atomic_numbers, pos, batch, edge_index, cell, cell_offsets,
            neighbors)


# speedup vs baseline: 4.2767x; 2.0965x over previous
import functools

import jax
import jax.numpy as jnp
from jax import lax
from jax.experimental import pallas as pl
from jax.experimental.pallas import tpu as pltpu
from jax.experimental.pallas import tpu_sc as plsc

N = 100000
H = 128

NC = 2
NS = 16
NW = NC * NS
CHUNK = 128
CPW = 25
NBUF = 7
LAG = 3
LAST = N - CHUNK


def _embed_sc(z, table):
    mesh = plsc.VectorSubcoreMesh(core_axis_name="c", subcore_axis_name="s")

    @functools.partial(
        pl.kernel,
        out_type=jax.ShapeDtypeStruct((N, H), jnp.float32),
        mesh=mesh,
        scratch_types=[
            pltpu.VMEM((CPW, CHUNK), jnp.int32),
            pltpu.VMEM((NBUF, CHUNK, H), jnp.float32),
            pltpu.VMEM_SHARED((100, H), jnp.float32),
            pltpu.SemaphoreType.DMA,
            pltpu.SemaphoreType.DMA((NBUF,)),
            pltpu.SemaphoreType.DMA((NBUF,)),
        ],
    )
    def k(z_hbm, table_hbm, out_hbm, idx_v, rows_v, table_sh, sem_i,
          sem_g, sem_w):
        sid = lax.axis_index("s")
        wid = sid * NC + lax.axis_index("c")

        @pl.when(sid == 0)
        def _():
            pltpu.sync_copy(table_hbm, table_sh)
        plsc.subcore_barrier()

        def off_of(j):
            return lax.min((wid + j * NW) * CHUNK, LAST)

        def fire_idx(j, c):
            pltpu.async_copy(z_hbm.at[pl.ds(off_of(j), CHUNK)],
                             idx_v.at[j], sem_i)
            return c
        lax.fori_loop(0, CPW, fire_idx, 0)

        def drain_idx(j, c):
            pltpu.make_async_copy(z_hbm.at[pl.ds(0, CHUNK)],
                                  idx_v.at[0], sem_i).wait()
            return c
        lax.fori_loop(0, CPW, drain_idx, 0)

        def gather(j, b):
            pltpu.async_copy(table_sh.at[idx_v.at[j]], rows_v.at[b],
                             sem_g.at[b])

        def write(j, b):
            pltpu.async_copy(rows_v.at[b],
                             out_hbm.at[pl.ds(off_of(j), CHUNK)],
                             sem_w.at[b])

        def wait_g(b):
            pltpu.make_async_copy(table_sh.at[idx_v.at[0]], rows_v.at[b],
                                  sem_g.at[b]).wait()

        def wait_w(b):
            pltpu.make_async_copy(rows_v.at[b],
                                  out_hbm.at[pl.ds(0, CHUNK)],
                                  sem_w.at[b]).wait()

        def body(j, c):
            b = lax.rem(j, NBUF)

            @pl.when(j >= NBUF)
            def _():
                wait_w(b)

            gather(j, b)

            @pl.when(j >= LAG)
            def _():
                bp = lax.rem(j - LAG, NBUF)
                wait_g(bp)
                write(j - LAG, bp)
            return c

        lax.fori_loop(0, CPW, body, 0)

        def tail(j, c):
            b = lax.rem(j, NBUF)
            wait_g(b)
            write(j, b)
            return c
        lax.fori_loop(CPW - LAG, CPW, tail, 0)

        def drain_w(j, c):
            wait_w(lax.rem(j, NBUF))
            return c
        lax.fori_loop(CPW - NBUF, CPW, drain_w, 0)

    return k(z, table)


def kernel(atomic_numbers, pos, batch, edge_index, cell, cell_offsets,
           neighbors, W):
    z = atomic_numbers.astype(jnp.int32)
    h = _embed_sc(z, W.astype(jnp.float32))
    return (h, atomic_numbers, pos, batch, edge_index, cell, cell_offsets,
            neighbors)
